# baseline (speedup 1.0000x reference)
import functools

import jax
import jax.numpy as jnp
from jax import lax
from jax.experimental import pallas as pl
from jax.experimental.pallas import tpu as pltpu
from jax.experimental.pallas import tpu_sc as plsc

N_CODEC = 8
D = 1024
T_TEXT = 256
T_PROM = 512
T_CODE = 1024
T = T_TEXT + 1 + T_PROM + 1 + T_CODE
B = 8
R = B * T
BAG = 9
C = 4
NBUF = 3
NUNIT = R // C
NW = 32
UPW = 114
UPW_ALLOC = 120
IROWS = UPW_ALLOC // 3
NSTORE = R // (2 * C)
SPW = UPW // 2
IW = 40

OFF_TEXT = N_CODEC * 1024
OFF_SEP = OFF_TEXT + 256
ZROW = OFF_SEP + 1
OFF_POS = ZROW + 1


def _emb_body(table, idx, out, idx_v, gbuf, obuf, sem):
    wid = lax.axis_index("s") * 2 + lax.axis_index("c")
    pltpu.sync_copy(idx.at[pl.ds(wid * IROWS, IROWS)], idx_v)

    def issue(row, off, u):
        pltpu.async_copy(table.at[idx_v.at[row, pl.ds(off, BAG * C)]],
                         gbuf.at[u], sem)

    def wait_one(u):
        pltpu.make_async_copy(table.at[idx_v.at[0, pl.ds(0, BAG * C)]],
                              gbuf.at[u], sem).wait()

    def reduce(u, half):
        for c in range(C):
            row = half * C + c

            @plsc.parallel_loop(0, D // 32, unroll=2)
            def _(j, c=c, row=row):
                jh = j >> 3
                dd = (j & 7) * 16
                we = gbuf[u, c, jh, pl.ds(dd, 16)]
                ae = lax.bitcast_convert_type(we << 16, jnp.float32)
                ao = lax.bitcast_convert_type(we & jnp.int32(-65536), jnp.float32)
                for s in range(1, BAG):
                    w = gbuf[u, s * C + c, jh, pl.ds(dd, 16)]
                    ae = ae + lax.bitcast_convert_type(w << 16, jnp.float32)
                    ao = ao + lax.bitcast_convert_type(w & jnp.int32(-65536), jnp.float32)
                base = jh * 128 + dd
                obuf[row, pl.ds(base, 16)] = ae
                obuf[row, pl.ds(D // 2 + base, 16)] = ao

    issue(0, 0, 0)
    issue(0, 40, 1)
    issue(0, 80, 2)

    def step(k, carry):
        g0 = 6 * k
        for m in range(6):
            gl = g0 + m
            wait_one(m % NBUF)
            reduce(m % NBUF, m % 2)

            @pl.when(gl + NBUF < UPW)
            def _():
                issue(2 * k + 1 + m // 3, (m % 3) * 40, m % NBUF)

            if m % 2 == 1:
                g8 = wid * SPW + 3 * k + m // 2

                @pl.when(g8 < NSTORE)
                def _():
                    pltpu.sync_copy(obuf, out.at[pl.ds(g8 * 2 * C, 2 * C)])

        return carry

    lax.fori_loop(0, UPW // 6, step, 0)


@functools.lru_cache(maxsize=1)
def _build_emb_kernel():
    mesh = plsc.VectorSubcoreMesh(core_axis_name="c", subcore_axis_name="s")
    return pl.kernel(
        _emb_body,
        mesh=mesh,
        out_type=jax.ShapeDtypeStruct((R, D), jnp.float32),
        scratch_types=[
            pltpu.VMEM((IROWS, 128), jnp.int32),
            pltpu.VMEM((NBUF, BAG * C, D // 256, 128), jnp.int32),
            pltpu.VMEM((2 * C, D), jnp.float32),
            pltpu.SemaphoreType.DMA,
        ],
    )


def kernel(text, prom, code, text_table, wave_tables, sep, pos_emb):
    i32 = jnp.int32
    f32 = jnp.float32
    wave_flat = wave_tables.astype(f32).reshape(N_CODEC * 1024, D)
    table = jnp.concatenate([
        wave_flat,
        text_table.astype(f32),
        sep.astype(f32),
        jnp.zeros((1, D), f32),
        pos_emb.astype(f32)[:T],
    ], axis=0)

    off = jnp.arange(N_CODEC, dtype=i32) * 1024
    prom_i = prom.astype(i32) + off
    code_i = code.astype(i32) + off
    text_i = jnp.concatenate(
        [text.astype(i32)[..., None] + OFF_TEXT,
         jnp.full((B, T_TEXT, N_CODEC - 1), ZROW, i32)], axis=-1)
    sep_i = jnp.concatenate(
        [jnp.full((B, 1, 1), OFF_SEP, i32),
         jnp.full((B, 1, N_CODEC - 1), ZROW, i32)], axis=-1)
    idx8 = jnp.concatenate([text_i, sep_i, prom_i, sep_i, code_i], axis=1)
    pos_i = jnp.broadcast_to(
        (OFF_POS + jnp.arange(T, dtype=i32))[None, :, None], (B, T, 1))
    idx9 = jnp.concatenate([idx8, pos_i], axis=2)
    idx9 = idx9.reshape(NUNIT, C, BAG).transpose(0, 2, 1).reshape(
        NUNIT, BAG * C)
    idx_units = jnp.concatenate(
        [idx9, jnp.full((NW * UPW - NUNIT, BAG * C), ZROW, i32)],
        axis=0).reshape(NW, UPW, BAG * C)
    idx_units = jnp.concatenate(
        [idx_units,
         jnp.full((NW, UPW_ALLOC - UPW, BAG * C), ZROW, i32)], axis=1)
    idx_pad = jnp.full((NW, IROWS, 3, 40), ZROW, i32)
    idx_pad = idx_pad.at[..., :BAG * C].set(
        idx_units.reshape(NW, IROWS, 3, BAG * C))
    idx_pad = jnp.concatenate(
        [idx_pad.reshape(NW, IROWS, 120),
         jnp.full((NW, IROWS, 8), ZROW, i32)], axis=-1)
    idx_pad = idx_pad.reshape(NW * IROWS, 128)

    table = table.astype(jnp.bfloat16).reshape(-1, 2, D // 2)
    table = jnp.swapaxes(table, 1, 2)
    table = lax.bitcast_convert_type(table, jnp.int32)
    table = table.reshape(-1, D // 256, 128)
    x = _build_emb_kernel()(table, idx_pad).reshape(B, T, D)
    l = jnp.full((B,), T, dtype=i32)
    return (x, l)


# speedup vs baseline: 2.2915x; 2.2915x over previous
import functools

import jax
import jax.numpy as jnp
from jax import lax
from jax.experimental import pallas as pl
from jax.experimental.pallas import tpu as pltpu
from jax.experimental.pallas import tpu_sc as plsc

N_CODEC = 8
D = 1024
T_TEXT = 256
T_PROM = 512
T_CODE = 1024
T = T_TEXT + 1 + T_PROM + 1 + T_CODE
B = 8
R = B * T
BAG = 9
C = 4
NBUF = 6
NPASS = 4
PW = D // (2 * NPASS)
NUNIT = R // C
NW = 32
UPW = 114
UPW_ALLOC = 120
IROWS = UPW_ALLOC // 3
NSTORE = R // (2 * C)
SPW = UPW // 2

OFF_TEXT = N_CODEC * 1024
OFF_SEP = OFF_TEXT + 256
ZROW = OFF_SEP + 1
OFF_POS = ZROW + 1
NROWS = OFF_POS + T
NR_PAD = 10368
STRIPE = NR_PAD // 16


def _emb_body(tbl, idx, out, idx_v, gbuf, obuf, shr, sem):
    sid = lax.axis_index("s")
    wid = sid * 2 + lax.axis_index("c")
    pltpu.sync_copy(idx.at[pl.ds(wid * IROWS, IROWS)], idx_v)

    def issue(row, off, u):
        pltpu.async_copy(shr.at[idx_v.at[row, pl.ds(off, BAG * C)]],
                         gbuf.at[u], sem)

    def wait_one(u):
        pltpu.make_async_copy(shr.at[idx_v.at[0, pl.ds(0, BAG * C)]],
                              gbuf.at[u], sem).wait()

    def reduce(u, half):
        for c in range(C):
            row = half * C + c

            @plsc.parallel_loop(0, PW // 16, unroll=4)
            def _(j, c=c, row=row):
                dd = j * 16
                we = gbuf[u, c, pl.ds(dd, 16)]
                ae = lax.bitcast_convert_type(we << 16, jnp.float32)
                ao = lax.bitcast_convert_type(we & jnp.int32(-65536),
                                              jnp.float32)
                for s in range(1, BAG):
                    w = gbuf[u, s * C + c, pl.ds(dd, 16)]
                    ae = ae + lax.bitcast_convert_type(w << 16, jnp.float32)
                    ao = ao + lax.bitcast_convert_type(
                        w & jnp.int32(-65536), jnp.float32)
                obuf[row, pl.ds(dd, 16)] = ae
                obuf[row, pl.ds(PW + dd, 16)] = ao

    def pass_body(p, carry):
        plsc.subcore_barrier()
        pltpu.sync_copy(tbl.at[p, pl.ds(sid * STRIPE, STRIPE)],
                        shr.at[pl.ds(sid * STRIPE, STRIPE)])
        plsc.subcore_barrier()

        issue(0, 0, 0)
        issue(0, 40, 1)
        issue(0, 80, 2)
        issue(1, 0, 3)
        issue(1, 40, 4)
        issue(1, 80, 5)

        def step(k, kcarry):
            for m in range(6):
                gl = 6 * k + m
                wait_one(m)
                reduce(m, m % 2)

                @pl.when(gl + NBUF < UPW)
                def _():
                    issue(2 * k + 2 + m // 3, (m % 3) * 40, m)

                if m % 2 == 1:
                    g8 = wid * SPW + 3 * k + m // 2

                    @pl.when(g8 < NSTORE)
                    def _():
                        pltpu.sync_copy(
                            obuf,
                            out.at[pl.ds(g8 * 2 * C, 2 * C),
                                   pl.ds(p * 2 * PW, 2 * PW)])
            return kcarry

        lax.fori_loop(0, UPW // 6, step, 0)
        return carry

    lax.fori_loop(0, NPASS, pass_body, 0)


@functools.lru_cache(maxsize=1)
def _build_emb_kernel():
    mesh = plsc.VectorSubcoreMesh(core_axis_name="c", subcore_axis_name="s")
    return pl.kernel(
        _emb_body,
        mesh=mesh,
        out_type=jax.ShapeDtypeStruct((R, D), jnp.float32),
        scratch_types=[
            pltpu.VMEM((IROWS, 128), jnp.int32),
            pltpu.VMEM((NBUF, BAG * C, PW), jnp.int32),
            pltpu.VMEM((2 * C, 2 * PW), jnp.float32),
            pltpu.VMEM_SHARED((NR_PAD, PW), jnp.int32),
            pltpu.SemaphoreType.DMA,
        ],
    )


def kernel(text, prom, code, text_table, wave_tables, sep, pos_emb):
    i32 = jnp.int32
    f32 = jnp.float32
    wave_flat = wave_tables.astype(f32).reshape(N_CODEC * 1024, D)
    table = jnp.concatenate([
        wave_flat,
        text_table.astype(f32),
        sep.astype(f32),
        jnp.zeros((1, D), f32),
        pos_emb.astype(f32)[:T],
        jnp.zeros((NR_PAD - NROWS, D), f32),
    ], axis=0)

    off = jnp.arange(N_CODEC, dtype=i32) * 1024
    prom_i = prom.astype(i32) + off
    code_i = code.astype(i32) + off
    text_i = jnp.concatenate(
        [text.astype(i32)[..., None] + OFF_TEXT,
         jnp.full((B, T_TEXT, N_CODEC - 1), ZROW, i32)], axis=-1)
    sep_i = jnp.concatenate(
        [jnp.full((B, 1, 1), OFF_SEP, i32),
         jnp.full((B, 1, N_CODEC - 1), ZROW, i32)], axis=-1)
    idx8 = jnp.concatenate([text_i, sep_i, prom_i, sep_i, code_i], axis=1)
    pos_i = jnp.broadcast_to(
        (OFF_POS + jnp.arange(T, dtype=i32))[None, :, None], (B, T, 1))
    idx9 = jnp.concatenate([idx8, pos_i], axis=2)
    idx9 = idx9.reshape(NUNIT, C, BAG).transpose(0, 2, 1).reshape(
        NUNIT, BAG * C)
    idx_units = jnp.concatenate(
        [idx9, jnp.full((NW * UPW - NUNIT, BAG * C), ZROW, i32)],
        axis=0).reshape(NW, UPW, BAG * C)
    idx_units = jnp.concatenate(
        [idx_units,
         jnp.full((NW, UPW_ALLOC - UPW, BAG * C), ZROW, i32)], axis=1)
    idx_pad = jnp.full((NW, IROWS, 3, 40), ZROW, i32)
    idx_pad = idx_pad.at[..., :BAG * C].set(
        idx_units.reshape(NW, IROWS, 3, BAG * C))
    idx_pad = jnp.concatenate(
        [idx_pad.reshape(NW, IROWS, 120),
         jnp.full((NW, IROWS, 8), ZROW, i32)], axis=-1)
    idx_pad = idx_pad.reshape(NW * IROWS, 128)

    tblp = table.astype(jnp.bfloat16).reshape(NR_PAD, NPASS, 2, PW)
    tblp = tblp.transpose(1, 0, 3, 2)
    tblp = lax.bitcast_convert_type(tblp, i32)

    x = _build_emb_kernel()(tblp, idx_pad).reshape(B, T, D)
    l = jnp.full((B,), T, dtype=i32)
    return (x, l)
